# baseline (speedup 1.0000x reference)
import functools

import jax
import jax.numpy as jnp
from jax.experimental import pallas as pl
from jax.experimental.pallas import tpu as pltpu

B = 64
L = 4096
H = L // 2
TI = 256
NI = L // TI


def _fused_kernel(predicts_ref, sims_ref, adjv_l, adjv_r, adjm_l, adjm_r,
                  out_ref, contrib, colsum, diag):
    i = pl.program_id(0)

    @pl.when(i == 0)
    def _init():
        contrib[...] = jnp.zeros_like(contrib)
        colsum[...] = jnp.zeros_like(colsum)

    p_i = predicts_ref[:, pl.ds(i * TI, TI)]
    ones8 = jnp.ones((8, TI), jnp.float32)

    m_l = adjm_l[...].astype(jnp.float32)
    a_l = adjv_l[...] * m_l
    contrib[:, 0:H] += jnp.dot(p_i, a_l, preferred_element_type=jnp.float32)
    colsum[:, 0:H] += jnp.dot(ones8, m_l, preferred_element_type=jnp.float32)

    m_r = adjm_r[...].astype(jnp.float32)
    a_r = adjv_r[...] * m_r
    contrib[:, H:L] += jnp.dot(p_i, a_r, preferred_element_type=jnp.float32)
    colsum[:, H:L] += jnp.dot(ones8, m_r, preferred_element_type=jnp.float32)

    rows = jax.lax.broadcasted_iota(jnp.int32, (TI, TI), 0)
    cols = jax.lax.broadcasted_iota(jnp.int32, (TI, TI), 1)
    eye = (rows == cols).astype(jnp.float32)

    @pl.when(i * TI < H)
    def _diag_left():
        m_sq = adjm_l[:, pl.ds(i * TI, TI)].astype(jnp.float32)
        diag[0:1, pl.ds(i * TI, TI)] = jnp.sum(m_sq * eye, axis=0,
                                               keepdims=True)

    @pl.when(i * TI >= H)
    def _diag_right():
        m_sq = adjm_r[:, pl.ds(i * TI - H, TI)].astype(jnp.float32)
        diag[0:1, pl.ds(i * TI, TI)] = jnp.sum(m_sq * eye, axis=0,
                                               keepdims=True)

    @pl.when(i == NI - 1)
    def _finalize():
        one_minus_diag = 1.0 - diag[...]
        relation = colsum[0:1, :] + one_minus_diag
        p = predicts_ref[...]
        cand = (contrib[...] + p * one_minus_diag) / relation
        res = p - jnp.dot(sims_ref[...], cand,
                          preferred_element_type=jnp.float32)
        sumsq = jnp.sum(res * res, axis=1, keepdims=True)
        norms = jnp.sqrt(sumsq)
        rowsum = jnp.sum(sims_ref[...], axis=1, keepdims=True)
        valid = (rowsum != 0.0).astype(jnp.float32)
        loss = jnp.sum(norms * valid) / jnp.sum(valid)
        out_ref[...] = jnp.reshape(loss, (1, 1))


@functools.partial(jax.jit, static_argnames=("interpret",))
def _run(predicts, similarities, adj_values, adj_mask, interpret=False):
    half = pl.BlockSpec((TI, H), lambda i: (i, 0))
    halfr = pl.BlockSpec((TI, H), lambda i: (i, 1))
    out = pl.pallas_call(
        _fused_kernel,
        grid=(NI,),
        in_specs=[
            pl.BlockSpec((B, L), lambda i: (0, 0)),
            pl.BlockSpec((B, B), lambda i: (0, 0)),
            half, halfr,
            half, halfr,
        ],
        out_specs=pl.BlockSpec((1, 1), lambda i: (0, 0)),
        out_shape=jax.ShapeDtypeStruct((1, 1), jnp.float32),
        scratch_shapes=[
            pltpu.VMEM((B, L), jnp.float32),
            pltpu.VMEM((8, L), jnp.float32),
            pltpu.VMEM((1, L), jnp.float32),
        ],
        interpret=interpret,
    )(predicts, similarities, adj_values, adj_values, adj_mask, adj_mask)
    return out[0, 0]


def kernel(predicts, similarities, adj_values, adj_mask):
    return _run(predicts, similarities, adj_values, adj_mask)
